# baseline (device time: 747327 ns/iter reference)
import jax
import jax.numpy as jnp
from jax import lax
from jax.experimental import pallas as pl
from jax.experimental.pallas import tpu as pltpu

N_DEV = 4
SQ = 2048
SKV_LOC = 2048
H_PER = 8
DH = 128
DMODEL = 1024
SLIVER = 256
KV_NEED = SKV_LOC + SLIVER
WIN = 128
SCALE = 0.08838834764831843
QB = 256
KB = 2 * QB
HALF = DMODEL // 2

_MESH = pl.DeviceIdType.MESH


def _kv_exchange(K_l, V_l, x2, Wq):

    def body(k_ref, v_ref, x_ref, wq_ref, kn_ref, vn_ref, q_ref,
             relay_ref, send_sems, recv_sems, copy_sems):
        my = lax.axis_index("i")

        def compute_q():
            q_ref[...] = jnp.dot(x_ref[...], wq_ref[...],
                                 preferred_element_type=jnp.float32)

        def recv(dst, sem_idx, dev):
            return pltpu.make_async_remote_copy(
                src_ref=dst, dst_ref=dst,
                send_sem=send_sems.at[7], recv_sem=recv_sems.at[sem_idx],
                device_id=(dev,), device_id_type=_MESH)

        @pl.when(my == 0)
        def _():
            rk = pltpu.make_async_remote_copy(
                src_ref=k_ref.at[:, pl.ds(2 * H_PER, H_PER), :],
                dst_ref=relay_ref,
                send_sem=send_sems.at[4], recv_sem=recv_sems.at[4],
                device_id=(1,), device_id_type=_MESH)
            rk.start()
            rv = pltpu.make_async_remote_copy(
                src_ref=v_ref.at[:, pl.ds(2 * H_PER, H_PER), :],
                dst_ref=relay_ref,
                send_sem=send_sems.at[5], recv_sem=recv_sems.at[5],
                device_id=(3,), device_id_type=_MESH)
            rv.start()
            mains = []
            for i, dev in enumerate((1, 3)):
                for t, (src, dst) in enumerate(((k_ref, kn_ref), (v_ref, vn_ref))):
                    r = pltpu.make_async_remote_copy(
                        src_ref=src.at[:, pl.ds(H_PER * dev, H_PER), :],
                        dst_ref=dst.at[pl.ds(0, SKV_LOC)],
                        send_sem=send_sems.at[2 * i + t],
                        recv_sem=recv_sems.at[t],
                        device_id=(dev,), device_id_type=_MESH)
                    r.start()
                    mains.append(r)
            copies = []
            for t, (src, dst) in enumerate(((k_ref, kn_ref), (v_ref, vn_ref))):
                c = pltpu.make_async_copy(
                    src.at[:, pl.ds(0, H_PER), :],
                    dst.at[pl.ds(0, SKV_LOC)], copy_sems.at[t])
                c.start()
                copies.append(c)
            compute_q()
            rk.wait_send()
            rv.wait_send()
            for r in mains:
                r.wait_send()
            for c in copies:
                c.wait()
            recv(kn_ref.at[pl.ds(SKV_LOC, SLIVER)], 2, 1).wait_recv()
            recv(vn_ref.at[pl.ds(SKV_LOC, SLIVER)], 3, 1).wait_recv()

        @pl.when(my == 1)
        def _():
            slivers = []
            for i, dev in enumerate((0, 2, 3)):
                for t, (src, dst) in enumerate(((k_ref, kn_ref), (v_ref, vn_ref))):
                    r = pltpu.make_async_remote_copy(
                        src_ref=src.at[pl.ds(0, SLIVER),
                                       pl.ds(H_PER * dev, H_PER), :],
                        dst_ref=dst.at[pl.ds(SKV_LOC, SLIVER)],
                        send_sem=send_sems.at[2 * i + t],
                        recv_sem=recv_sems.at[2 + t],
                        device_id=(dev,), device_id_type=_MESH)
                    r.start()
                    slivers.append(r)
            copies = []
            for t, (src, dst) in enumerate(((k_ref, kn_ref), (v_ref, vn_ref))):
                c = pltpu.make_async_copy(
                    src.at[pl.ds(0, SLIVER), pl.ds(H_PER, H_PER), :],
                    dst.at[pl.ds(SKV_LOC, SLIVER)], copy_sems.at[t])
                c.start()
                copies.append(c)
            compute_q()
            recv(relay_ref, 4, 0).wait_recv()
            fwd = pltpu.make_async_remote_copy(
                src_ref=relay_ref,
                dst_ref=kn_ref.at[pl.ds(0, SKV_LOC)],
                send_sem=send_sems.at[6], recv_sem=recv_sems.at[0],
                device_id=(2,), device_id_type=_MESH)
            fwd.start()
            for r in slivers:
                r.wait_send()
            for c in copies:
                c.wait()
            recv(kn_ref.at[pl.ds(0, SKV_LOC)], 0, 0).wait_recv()
            recv(vn_ref.at[pl.ds(0, SKV_LOC)], 1, 0).wait_recv()
            fwd.wait_send()

        @pl.when(my == 2)
        def _():
            compute_q()
            recv(kn_ref.at[pl.ds(0, SKV_LOC)], 0, 1).wait_recv()
            recv(vn_ref.at[pl.ds(0, SKV_LOC)], 1, 3).wait_recv()
            recv(kn_ref.at[pl.ds(SKV_LOC, SLIVER)], 2, 1).wait_recv()
            recv(vn_ref.at[pl.ds(SKV_LOC, SLIVER)], 3, 1).wait_recv()

        @pl.when(my == 3)
        def _():
            compute_q()
            recv(relay_ref, 5, 0).wait_recv()
            fwd = pltpu.make_async_remote_copy(
                src_ref=relay_ref,
                dst_ref=vn_ref.at[pl.ds(0, SKV_LOC)],
                send_sem=send_sems.at[6], recv_sem=recv_sems.at[1],
                device_id=(2,), device_id_type=_MESH)
            fwd.start()
            recv(kn_ref.at[pl.ds(0, SKV_LOC)], 0, 0).wait_recv()
            recv(vn_ref.at[pl.ds(0, SKV_LOC)], 1, 0).wait_recv()
            recv(kn_ref.at[pl.ds(SKV_LOC, SLIVER)], 2, 1).wait_recv()
            recv(vn_ref.at[pl.ds(SKV_LOC, SLIVER)], 3, 1).wait_recv()
            fwd.wait_send()

    return pl.pallas_call(
        body,
        out_shape=(
            jax.ShapeDtypeStruct((KV_NEED, H_PER, DH), jnp.float32),
            jax.ShapeDtypeStruct((KV_NEED, H_PER, DH), jnp.float32),
            jax.ShapeDtypeStruct((SQ, DMODEL), jnp.float32),
        ),
        in_specs=[
            pl.BlockSpec(memory_space=pl.ANY),
            pl.BlockSpec(memory_space=pl.ANY),
            pl.BlockSpec(memory_space=pltpu.MemorySpace.VMEM),
            pl.BlockSpec(memory_space=pltpu.MemorySpace.VMEM),
        ],
        out_specs=(
            pl.BlockSpec(memory_space=pl.ANY),
            pl.BlockSpec(memory_space=pl.ANY),
            pl.BlockSpec(memory_space=pltpu.MemorySpace.VMEM),
        ),
        scratch_shapes=[
            pltpu.VMEM((SKV_LOC, H_PER, DH), jnp.float32),
            pltpu.SemaphoreType.DMA((8,)),
            pltpu.SemaphoreType.DMA((6,)),
            pltpu.SemaphoreType.DMA((2,)),
        ],
    )(K_l, V_l, x2, Wq)


def _allgather_sum(p):

    def body(x_ref, out_ref, copy_sems, rs_sems, rr_sems, ls_sems, lr_sems):
        my = lax.axis_index("i")
        right = (my + 1) % N_DEV
        left = (my + N_DEV - 1) % N_DEV
        c0 = pltpu.make_async_copy(
            x_ref.at[:, pl.ds(0, HALF)],
            out_ref.at[0, :, pl.ds(0, HALF)], copy_sems.at[0])
        c1 = pltpu.make_async_copy(
            x_ref.at[:, pl.ds(HALF, HALF)],
            out_ref.at[0, :, pl.ds(HALF, HALF)], copy_sems.at[1])
        c0.start()
        c1.start()
        c0.wait()
        c1.wait()
        for h in range(N_DEV - 1):
            r = pltpu.make_async_remote_copy(
                src_ref=out_ref.at[h, :, pl.ds(0, HALF)],
                dst_ref=out_ref.at[h + 1, :, pl.ds(0, HALF)],
                send_sem=rs_sems.at[h], recv_sem=rr_sems.at[h],
                device_id=(right,), device_id_type=_MESH)
            l = pltpu.make_async_remote_copy(
                src_ref=out_ref.at[h, :, pl.ds(HALF, HALF)],
                dst_ref=out_ref.at[h + 1, :, pl.ds(HALF, HALF)],
                send_sem=ls_sems.at[h], recv_sem=lr_sems.at[h],
                device_id=(left,), device_id_type=_MESH)
            r.start()
            l.start()
            r.wait()
            l.wait()

    return pl.pallas_call(
        body,
        out_shape=jax.ShapeDtypeStruct((N_DEV, SQ, DMODEL), jnp.float32),
        in_specs=[pl.BlockSpec(memory_space=pl.ANY)],
        out_specs=pl.BlockSpec(memory_space=pl.ANY),
        scratch_shapes=[
            pltpu.SemaphoreType.DMA((2,)),
            pltpu.SemaphoreType.DMA((N_DEV - 1,)),
            pltpu.SemaphoreType.DMA((N_DEV - 1,)),
            pltpu.SemaphoreType.DMA((N_DEV - 1,)),
            pltpu.SemaphoreType.DMA((N_DEV - 1,)),
        ],
    )(p)


def kernel(x, Wq, K_ext, V_ext, Wo):
    x2 = x[0]
    K_l = K_ext[0]
    V_l = V_ext[0]

    K_n, V_n, Qf = _kv_exchange(K_l, V_l, x2, Wq)
    Q = Qf.reshape(SQ, H_PER, DH)

    pad = jnp.zeros((WIN, H_PER, DH), jnp.float32)
    K_p = jnp.concatenate([pad, K_n], axis=0)
    V_p = jnp.concatenate([pad, V_n], axis=0)

    q_idx = jnp.arange(QB)[:, None]
    j_idx = jnp.arange(KB)[None, :]
    base_mask = jnp.abs(q_idx - j_idx + WIN) <= WIN
    ctx_blocks = []
    for b in range(SQ // QB):
        Qb = Q[b * QB:(b + 1) * QB]
        Kb = K_p[b * QB: b * QB + KB]
        Vb = V_p[b * QB: b * QB + KB]
        m = base_mask
        if b == 0:
            m = m & (j_idx >= WIN)
        s = jnp.einsum("qhd,khd->hqk", Qb, Kb,
                       preferred_element_type=jnp.float32) * SCALE
        s = jnp.where(m[None], s, -1e9)
        s = s - s.max(axis=-1, keepdims=True)
        w = jnp.exp(s)
        w = w / w.sum(axis=-1, keepdims=True)
        ctx_blocks.append(jnp.einsum("hqk,khd->qhd", w, Vb,
                                     preferred_element_type=jnp.float32))
    ctx = jnp.concatenate(ctx_blocks, axis=0).reshape(SQ, H_PER * DH)
    partial = ctx @ Wo

    gathered = _allgather_sum(partial)
    out = gathered.sum(axis=0)
    return out[None]


# device time: 529181 ns/iter; 1.4122x vs baseline; 1.4122x over previous
import jax
import jax.numpy as jnp
from jax import lax
from jax.experimental import pallas as pl
from jax.experimental.pallas import tpu as pltpu

N_DEV = 4
SQ = 2048
SKV_LOC = 2048
H_PER = 8
DH = 128
DMODEL = 1024
SLIVER = 256
KV_NEED = SKV_LOC + SLIVER
WIN = 128
SCALE = 0.08838834764831843
QB = 256
KB = 2 * QB
HALF = DMODEL // 2

_MESH = pl.DeviceIdType.MESH


def _kv_exchange(K_l, V_l, x2, Wq):

    def body(k_ref, v_ref, x_ref, wq_ref, kn_ref, vn_ref, q_ref,
             relay_ref, send_sems, recv_sems):
        my = lax.axis_index("i")

        def compute_q():
            q_ref[...] = jnp.dot(x_ref[...], wq_ref[...],
                                 preferred_element_type=jnp.float32)

        def recv(dst, sem_idx, dev):
            return pltpu.make_async_remote_copy(
                src_ref=dst, dst_ref=dst,
                send_sem=send_sems.at[7], recv_sem=recv_sems.at[sem_idx],
                device_id=(dev,), device_id_type=_MESH)

        @pl.when(my == 0)
        def _():
            rk = pltpu.make_async_remote_copy(
                src_ref=k_ref.at[:, pl.ds(2 * H_PER, H_PER), :],
                dst_ref=relay_ref,
                send_sem=send_sems.at[4], recv_sem=recv_sems.at[4],
                device_id=(1,), device_id_type=_MESH)
            rk.start()
            rv = pltpu.make_async_remote_copy(
                src_ref=v_ref.at[:, pl.ds(2 * H_PER, H_PER), :],
                dst_ref=relay_ref,
                send_sem=send_sems.at[5], recv_sem=recv_sems.at[5],
                device_id=(3,), device_id_type=_MESH)
            rv.start()
            mains = []
            for i, dev in enumerate((1, 3)):
                for t, (src, dst) in enumerate(((k_ref, kn_ref), (v_ref, vn_ref))):
                    r = pltpu.make_async_remote_copy(
                        src_ref=src.at[:, pl.ds(H_PER * dev, H_PER), :],
                        dst_ref=dst.at[pl.ds(0, SKV_LOC)],
                        send_sem=send_sems.at[2 * i + t],
                        recv_sem=recv_sems.at[t],
                        device_id=(dev,), device_id_type=_MESH)
                    r.start()
                    mains.append(r)
            compute_q()
            rk.wait_send()
            rv.wait_send()
            for r in mains:
                r.wait_send()
            recv(kn_ref.at[pl.ds(SKV_LOC, SLIVER)], 2, 1).wait_recv()
            recv(vn_ref.at[pl.ds(SKV_LOC, SLIVER)], 3, 1).wait_recv()

        @pl.when(my == 1)
        def _():
            slivers = []
            for i, dev in enumerate((0, 2, 3)):
                for t, (src, dst) in enumerate(((k_ref, kn_ref), (v_ref, vn_ref))):
                    r = pltpu.make_async_remote_copy(
                        src_ref=src.at[pl.ds(0, SLIVER),
                                       pl.ds(H_PER * dev, H_PER), :],
                        dst_ref=dst.at[pl.ds(SKV_LOC, SLIVER)],
                        send_sem=send_sems.at[2 * i + t],
                        recv_sem=recv_sems.at[2 + t],
                        device_id=(dev,), device_id_type=_MESH)
                    r.start()
                    slivers.append(r)
            compute_q()
            recv(relay_ref, 4, 0).wait_recv()
            fwd = pltpu.make_async_remote_copy(
                src_ref=relay_ref,
                dst_ref=kn_ref.at[pl.ds(0, SKV_LOC)],
                send_sem=send_sems.at[6], recv_sem=recv_sems.at[0],
                device_id=(2,), device_id_type=_MESH)
            fwd.start()
            for r in slivers:
                r.wait_send()
            recv(kn_ref.at[pl.ds(0, SKV_LOC)], 0, 0).wait_recv()
            recv(vn_ref.at[pl.ds(0, SKV_LOC)], 1, 0).wait_recv()
            fwd.wait_send()

        @pl.when(my == 2)
        def _():
            compute_q()
            recv(kn_ref.at[pl.ds(0, SKV_LOC)], 0, 1).wait_recv()
            recv(vn_ref.at[pl.ds(0, SKV_LOC)], 1, 3).wait_recv()
            recv(kn_ref.at[pl.ds(SKV_LOC, SLIVER)], 2, 1).wait_recv()
            recv(vn_ref.at[pl.ds(SKV_LOC, SLIVER)], 3, 1).wait_recv()

        @pl.when(my == 3)
        def _():
            compute_q()
            recv(relay_ref, 5, 0).wait_recv()
            fwd = pltpu.make_async_remote_copy(
                src_ref=relay_ref,
                dst_ref=vn_ref.at[pl.ds(0, SKV_LOC)],
                send_sem=send_sems.at[6], recv_sem=recv_sems.at[1],
                device_id=(2,), device_id_type=_MESH)
            fwd.start()
            recv(kn_ref.at[pl.ds(0, SKV_LOC)], 0, 0).wait_recv()
            recv(vn_ref.at[pl.ds(0, SKV_LOC)], 1, 0).wait_recv()
            recv(kn_ref.at[pl.ds(SKV_LOC, SLIVER)], 2, 1).wait_recv()
            recv(vn_ref.at[pl.ds(SKV_LOC, SLIVER)], 3, 1).wait_recv()
            fwd.wait_send()

    return pl.pallas_call(
        body,
        out_shape=(
            jax.ShapeDtypeStruct((KV_NEED, H_PER, DH), jnp.float32),
            jax.ShapeDtypeStruct((KV_NEED, H_PER, DH), jnp.float32),
            jax.ShapeDtypeStruct((SQ, DMODEL), jnp.float32),
        ),
        in_specs=[
            pl.BlockSpec(memory_space=pl.ANY),
            pl.BlockSpec(memory_space=pl.ANY),
            pl.BlockSpec(memory_space=pltpu.MemorySpace.VMEM),
            pl.BlockSpec(memory_space=pltpu.MemorySpace.VMEM),
        ],
        out_specs=(
            pl.BlockSpec(memory_space=pl.ANY),
            pl.BlockSpec(memory_space=pl.ANY),
            pl.BlockSpec(memory_space=pltpu.MemorySpace.VMEM),
        ),
        scratch_shapes=[
            pltpu.VMEM((SKV_LOC, H_PER, DH), jnp.float32),
            pltpu.SemaphoreType.DMA((8,)),
            pltpu.SemaphoreType.DMA((6,)),
        ],
    )(K_l, V_l, x2, Wq)


def _allgather_sum(p):

    def body(x_ref, out_ref, copy_sems, rs_sems, rr_sems, ls_sems, lr_sems):
        my = lax.axis_index("i")
        right = (my + 1) % N_DEV
        left = (my + N_DEV - 1) % N_DEV
        c0 = pltpu.make_async_copy(
            x_ref.at[:, pl.ds(0, HALF)],
            out_ref.at[0, :, pl.ds(0, HALF)], copy_sems.at[0])
        c1 = pltpu.make_async_copy(
            x_ref.at[:, pl.ds(HALF, HALF)],
            out_ref.at[0, :, pl.ds(HALF, HALF)], copy_sems.at[1])
        c0.start()
        c1.start()
        c0.wait()
        c1.wait()
        for h in range(N_DEV - 1):
            r = pltpu.make_async_remote_copy(
                src_ref=out_ref.at[h, :, pl.ds(0, HALF)],
                dst_ref=out_ref.at[h + 1, :, pl.ds(0, HALF)],
                send_sem=rs_sems.at[h], recv_sem=rr_sems.at[h],
                device_id=(right,), device_id_type=_MESH)
            l = pltpu.make_async_remote_copy(
                src_ref=out_ref.at[h, :, pl.ds(HALF, HALF)],
                dst_ref=out_ref.at[h + 1, :, pl.ds(HALF, HALF)],
                send_sem=ls_sems.at[h], recv_sem=lr_sems.at[h],
                device_id=(left,), device_id_type=_MESH)
            r.start()
            l.start()
            r.wait()
            l.wait()

    return pl.pallas_call(
        body,
        out_shape=jax.ShapeDtypeStruct((N_DEV, SQ, DMODEL), jnp.float32),
        in_specs=[pl.BlockSpec(memory_space=pl.ANY)],
        out_specs=pl.BlockSpec(memory_space=pl.ANY),
        scratch_shapes=[
            pltpu.SemaphoreType.DMA((2,)),
            pltpu.SemaphoreType.DMA((N_DEV - 1,)),
            pltpu.SemaphoreType.DMA((N_DEV - 1,)),
            pltpu.SemaphoreType.DMA((N_DEV - 1,)),
            pltpu.SemaphoreType.DMA((N_DEV - 1,)),
        ],
    )(p)


def kernel(x, Wq, K_ext, V_ext, Wo):
    x2 = x[0]
    K_l = K_ext[0]
    V_l = V_ext[0]

    K_n, V_n, Qf = _kv_exchange(K_l, V_l, x2, Wq)
    Q = Qf.reshape(SQ, H_PER, DH)

    idx = lax.axis_index("i")
    K_own = lax.dynamic_slice_in_dim(K_l, idx * H_PER, H_PER, axis=1)
    V_own = lax.dynamic_slice_in_dim(V_l, idx * H_PER, H_PER, axis=1)
    K_n = jnp.concatenate([
        jnp.where(idx == 0, K_own, K_n[:SKV_LOC]),
        jnp.where(idx == 1, K_own[:SLIVER], K_n[SKV_LOC:]),
    ], axis=0)
    V_n = jnp.concatenate([
        jnp.where(idx == 0, V_own, V_n[:SKV_LOC]),
        jnp.where(idx == 1, V_own[:SLIVER], V_n[SKV_LOC:]),
    ], axis=0)

    pad = jnp.zeros((WIN, H_PER, DH), jnp.float32)
    K_p = jnp.concatenate([pad, K_n], axis=0)
    V_p = jnp.concatenate([pad, V_n], axis=0)

    q_idx = jnp.arange(QB)[:, None]
    j_idx = jnp.arange(KB)[None, :]
    base_mask = jnp.abs(q_idx - j_idx + WIN) <= WIN
    ctx_blocks = []
    for b in range(SQ // QB):
        Qb = Q[b * QB:(b + 1) * QB]
        Kb = K_p[b * QB: b * QB + KB]
        Vb = V_p[b * QB: b * QB + KB]
        m = base_mask
        if b == 0:
            m = m & (j_idx >= WIN)
        s = jnp.einsum("qhd,khd->hqk", Qb, Kb,
                       preferred_element_type=jnp.float32) * SCALE
        s = jnp.where(m[None], s, -1e9)
        s = s - s.max(axis=-1, keepdims=True)
        w = jnp.exp(s)
        w = w / w.sum(axis=-1, keepdims=True)
        ctx_blocks.append(jnp.einsum("hqk,khd->qhd", w, Vb,
                                     preferred_element_type=jnp.float32))
    ctx = jnp.concatenate(ctx_blocks, axis=0).reshape(SQ, H_PER * DH)
    partial = ctx @ Wo

    gathered = _allgather_sum(partial)
    out = gathered.sum(axis=0)
    return out[None]
